# baseline (speedup 1.0000x reference)
import functools

import jax
import jax.numpy as jnp
from jax import lax
from jax.experimental import pallas as pl
from jax.experimental.pallas import tpu as pltpu
from jax.experimental.pallas import tpu_sc as plsc

DIM = 64
PAD_DIM = 128
NUM_CORES = 2
NUM_SUBCORES = 16
NW = NUM_CORES * NUM_SUBCORES


@functools.partial(jax.jit, static_argnames=("chunk", "nbuf", "s0", "s1"))
def _emb_lookup(idx_flat, table_pad, s0, s1, chunk=400, nbuf=2):
    B = idx_flat.shape[0]
    per_w = B // NW
    n_chunks = per_w // chunk
    assert n_chunks % nbuf == 0 and n_chunks > nbuf
    mesh = plsc.VectorSubcoreMesh(core_axis_name="c", subcore_axis_name="s")

    @functools.partial(
        pl.kernel,
        mesh=mesh,
        out_type=jax.ShapeDtypeStruct((s0 * s1, DIM), jnp.float32),
        scratch_types=[
            pltpu.VMEM((per_w,), jnp.int32),
            pltpu.VMEM((nbuf, chunk, PAD_DIM), jnp.float32),
            [pltpu.SemaphoreType.DMA] * nbuf,
            [pltpu.SemaphoreType.DMA] * nbuf,
        ],
        compiler_params=pltpu.CompilerParams(use_tc_tiling_on_sc=False),
    )
    def k(idx_hbm, table_hbm, out_hbm, idx_v, rows_v, gsems, ssems):
        wid = lax.axis_index("s") * NUM_CORES + lax.axis_index("c")
        base = wid * per_w
        out_flat = out_hbm

        def gather(g, b):
            return pltpu.make_async_copy(
                table_hbm.at[idx_v.at[pl.ds(g * chunk, chunk)]],
                rows_v.at[b],
                gsems[b],
            )

        def store(g, b):
            return pltpu.make_async_copy(
                rows_v.at[b, :, pl.ds(0, DIM)],
                out_flat.at[pl.ds(base + g * chunk, chunk)],
                ssems[b],
            )

        pltpu.sync_copy(idx_hbm.at[pl.ds(base, per_w)], idx_v)

        for b in range(nbuf):
            gather(b, b).start()

        @pl.loop(0, n_chunks - nbuf, step=nbuf)
        def _(g0):
            for b in range(nbuf):
                g = g0 + b
                gather(g, b).wait()
                store(g, b).start()
                store(g, b).wait()
                gather(g + nbuf, b).start()

        for b in range(nbuf):
            g = n_chunks - nbuf + b
            gather(g, b).wait()
            store(g, b).start()
        for b in range(nbuf):
            g = n_chunks - nbuf + b
            store(g, b).wait()

    return k(idx_flat, table_pad)


def kernel(x, table):
    S0, S1 = x.shape
    V = table.shape[0]
    vpad = -V % 8
    idx_flat = x.reshape(-1).astype(jnp.int32)
    table_pad = jnp.pad(table, ((0, vpad), (0, PAD_DIM - DIM)))
    out = _emb_lookup(idx_flat, table_pad, S0, S1)
    return out.reshape(S0, S1, DIM)


# speedup vs baseline: 1.0067x; 1.0067x over previous
import functools

import jax
import jax.numpy as jnp
from jax import lax
from jax.experimental import pallas as pl
from jax.experimental.pallas import tpu as pltpu
from jax.experimental.pallas import tpu_sc as plsc

DIM = 64
NUM_CORES = 2
NUM_SUBCORES = 16
NW = NUM_CORES * NUM_SUBCORES


@functools.partial(jax.jit, static_argnames=("rpc", "nbuf"))
def _emb_lookup(x, table, rpc=8, nbuf=2):
    s0, s1 = x.shape
    rows_w = s0 // NW
    n_chunks = rows_w // rpc
    chunk = rpc * s1
    assert n_chunks % nbuf == 0 and n_chunks > nbuf
    mesh = plsc.VectorSubcoreMesh(core_axis_name="c", subcore_axis_name="s")

    @functools.partial(
        pl.kernel,
        mesh=mesh,
        out_type=jax.ShapeDtypeStruct((s0, s1, DIM), jnp.float32),
        scratch_types=[
            pltpu.VMEM((rows_w, s1), jnp.int32),
            pltpu.VMEM((nbuf, rpc, s1, DIM), jnp.float32),
            [pltpu.SemaphoreType.DMA] * nbuf,
            [pltpu.SemaphoreType.DMA] * nbuf,
        ],
        compiler_params=pltpu.CompilerParams(use_tc_tiling_on_sc=False),
    )
    def k(idx_hbm, table_hbm, out_hbm, idx_v, rows_v, gsems, ssems):
        wid = lax.axis_index("s") * NUM_CORES + lax.axis_index("c")
        row_base = wid * rows_w

        def gather_start(g, b):
            for j in range(rpc):
                pltpu.make_async_copy(
                    table_hbm.at[idx_v.at[g * rpc + j]],
                    rows_v.at[b, j],
                    gsems[b],
                ).start()

        def gather_wait(g, b):
            pltpu.make_async_copy(
                out_hbm.at[pl.ds(row_base + g * rpc, rpc)],
                rows_v.at[b],
                gsems[b],
            ).wait()

        def store(g, b):
            return pltpu.make_async_copy(
                rows_v.at[b],
                out_hbm.at[pl.ds(row_base + g * rpc, rpc)],
                ssems[b],
            )

        pltpu.sync_copy(idx_hbm.at[pl.ds(row_base, rows_w)], idx_v)

        for b in range(nbuf):
            gather_start(b, b)

        @pl.loop(0, n_chunks - nbuf, step=nbuf)
        def _(g0):
            for b in range(nbuf):
                g = g0 + b
                gather_wait(g, b)
                store(g, b).start()
                store(g, b).wait()
                gather_start(g + nbuf, b)

        for b in range(nbuf):
            g = n_chunks - nbuf + b
            gather_wait(g, b)
            store(g, b).start()
        for b in range(nbuf):
            g = n_chunks - nbuf + b
            store(g, b).wait()

    return k(x, table)


def kernel(x, table):
    return _emb_lookup(x.astype(jnp.int32), table)
